# baseline (speedup 1.0000x reference)
import functools

import jax
import jax.numpy as jnp
from jax import lax
from jax.experimental import pallas as pl
from jax.experimental.pallas import tpu as pltpu
from jax.experimental.pallas import tpu_sc as plsc

N = 100000
E = 1600000
T = 50
HID = 256

NS = 16
NPT = 6272
N_PAD = NS * NPT
EPT = 100352
E_PAD = NS * EPT
CH = 7168
NCH = EPT // CH
NV = NPT // 16

NB = 3136
N_GRID = N_PAD // NB


def _std_body(pT_ref, out_ref):
    x = pT_ref[...]
    s1 = jnp.sum(x, axis=1)
    s2 = jnp.sum(x * x, axis=1)
    out_ref[...] = jnp.pad(jnp.stack([s1, s2]), ((0, 0), (0, 125)))


def _col_stats(params_T):
    return pl.pallas_call(
        _std_body,
        out_shape=jax.ShapeDtypeStruct((2, 128), jnp.float32),
    )(params_T)


def _mlp_body(p_ref, w1_ref, b1_ref, w2_ref, b2_ref, w3_ref, b3_ref,
              stat_ref, k_ref, a_ref, b_ref):
    i = pl.program_id(0)
    s1 = stat_ref[0:1, 0:3] * (1.0 / N)
    s2 = stat_ref[1:2, 0:3] * (1.0 / N)
    stds = jnp.sqrt(jnp.maximum(s2 - s1 * s1, 0.0))
    lane = lax.broadcasted_iota(jnp.int32, (1, 3), 1)
    stds = jnp.where(lane == 0, 1.0, stds)
    w1s = w1_ref[...] / stds

    pn = p_ref[...]
    h = lax.dot_general(pn, w1s, (((1,), (1,)), ((), ())),
                        preferred_element_type=jnp.float32)
    h = jnp.maximum(h + b1_ref[...], 0.0)
    h = lax.dot_general(h, w2_ref[...], (((1,), (1,)), ((), ())),
                        preferred_element_type=jnp.float32)
    h = jnp.maximum(h + b2_ref[...], 0.0)
    raw = lax.dot_general(h, w3_ref[...], (((1,), (1,)), ((), ())),
                          preferred_element_type=jnp.float32)
    raw = raw + b3_ref[...]

    row = i * NB + lax.broadcasted_iota(jnp.int32, (NB, 1), 0)
    valid = row < N
    p0 = jax.nn.sigmoid(raw[:, 0:1])
    p1 = jax.nn.sigmoid(raw[:, 1:2] - 3.0)
    kk = (p0 * 0.25 + 0.005) * (1.0 / 16.0)
    xw = jnp.clip(p1 * 1.2, 0.0, 0.95)
    k_ref[...] = jnp.where(valid, kk, 0.0)
    a_ref[...] = jnp.where(valid, xw, 0.0)
    b_ref[...] = jnp.where(valid, 1.0 - xw, 0.0)


def _coeffs(params_phys, W1, b1, W2, b2, W3, b3):
    stats = _col_stats(params_phys.T)
    out_spec = pl.BlockSpec((NB, 1), lambda i: (i, 0))
    full = lambda *s: pl.BlockSpec(s, lambda i: tuple(0 for _ in s))
    k16, a, b = pl.pallas_call(
        _mlp_body,
        grid=(N_GRID,),
        in_specs=[
            pl.BlockSpec((NB, 3), lambda i: (i, 0)),
            full(HID, 3), full(1, HID), full(HID, HID), full(1, HID),
            full(2, HID), full(1, 2), full(2, 128),
        ],
        out_specs=[out_spec, out_spec, out_spec],
        out_shape=[jax.ShapeDtypeStruct((N_PAD, 1), jnp.float32)] * 3,
    )(params_phys, W1, b1.reshape(1, HID), W2, b2.reshape(1, HID),
      W3, b3.reshape(1, 2), stats)
    return k16.reshape(N_PAD), a.reshape(N_PAD), b.reshape(N_PAD)


def _scan_body(x_hbm, k_hbm, a_hbm, b_hbm, src_hbm, dst_hbm, out_hbm,
               contrib, inflow, k16, av, bv, st, xb, cb, ib, zb,
               sbuf0, sbuf1, dbuf0, dbuf1, dbuf2, mbuf0, mbuf1,
               sem_s0, sem_s1, sem_d0, sem_d1, sem_d2, sem_g, sem_sc,
               sem_x, sem_o):
    sbuf = (sbuf0, sbuf1)
    dbuf = (dbuf0, dbuf1, dbuf2)
    mbuf = (mbuf0, mbuf1)
    sem_s = (sem_s0, sem_s1)
    sem_d = (sem_d0, sem_d1, sem_d2)
    s = lax.axis_index("s")
    nbase = s * NPT
    ebase = s * EPT
    nsl = pl.ds(nbase, NPT)

    pltpu.sync_copy(k_hbm.at[nsl], k16)
    pltpu.sync_copy(a_hbm.at[nsl], av)
    pltpu.sync_copy(b_hbm.at[nsl], bv)

    def zero_i(i, carry):
        z = jnp.zeros((16,), jnp.float32)
        st[pl.ds(i * 16, 16)] = z
        cb[pl.ds(i * 16, 16)] = z
        zb[pl.ds(i * 16, 16)] = z
        return carry

    lax.fori_loop(0, NV, zero_i, 0)

    def start_idx(t, i):
        eb = ebase + i * CH
        pltpu.async_copy(src_hbm.at[pl.ds(eb, CH)], sbuf[i % 2],
                         sem_s[i % 2])
        pltpu.async_copy(dst_hbm.at[pl.ds(eb, CH)], dbuf[i % 3],
                         sem_d[i % 3])

    def wait_idx(i):
        eb = pl.ds(0, CH)
        pltpu.make_async_copy(src_hbm.at[eb], sbuf[i % 2],
                              sem_s[i % 2]).wait()
        pltpu.make_async_copy(dst_hbm.at[eb], dbuf[i % 3],
                              sem_d[i % 3]).wait()

    def start_gather(i):
        return pltpu.async_copy(contrib.at[sbuf[i % 2]], mbuf[i % 2],
                                sem_g)

    def start_scatter(i):
        return pltpu.async_copy(mbuf[i % 2], inflow.at[dbuf[i % 3]],
                                sem_sc, add=True)

    def step(t, carry):
        start_idx(t, 0)
        start_idx(t, 1)
        cpx = pltpu.async_copy(x_hbm.at[t, nsl], xb, sem_x)
        pltpu.sync_copy(cb, contrib.at[nsl])
        pltpu.sync_copy(zb, inflow.at[nsl])
        plsc.subcore_barrier()

        wait_idx(0)
        g = start_gather(0)
        sc = None
        for i in range(NCH):
            g.wait()
            if sc is not None:
                sc.wait()
            sc = start_scatter(i)
            if i + 1 < NCH:
                wait_idx(i + 1)
                g = start_gather(i + 1)
            if i + 2 < NCH:
                start_idx(t, i + 2)
        sc.wait()
        plsc.subcore_barrier()

        pltpu.sync_copy(inflow.at[nsl], ib)
        cpx.wait()

        @pl.when(t > 0)
        def _():
            pltpu.make_async_copy(st, out_hbm.at[t - 1, nsl], sem_o).wait()

        def update_i(i, c):
            for u in range(4):
                sl = pl.ds((i * 4 + u) * 16, 16)
                xt = xb[sl] * (1.0 / 86400.0)
                ns = av[sl] * st[sl] + bv[sl] * (ib[sl] + xt)
                st[sl] = ns
                cb[sl] = ns * k16[sl]
            return c

        lax.fori_loop(0, NV // 4, update_i, 0)
        pltpu.async_copy(st, out_hbm.at[t, nsl], sem_o)
        return carry

    lax.fori_loop(0, T, step, 0)
    pltpu.make_async_copy(st, out_hbm.at[T - 1, nsl], sem_o).wait()


_scan_kernel = functools.partial(
    pl.kernel,
    out_type=jax.ShapeDtypeStruct((T, N_PAD), jnp.float32),
    mesh=plsc.VectorSubcoreMesh(
        core_axis_name="c", subcore_axis_name="s", num_cores=1),
    scratch_types=[
        pltpu.VMEM_SHARED((N_PAD,), jnp.float32),
        pltpu.VMEM_SHARED((N_PAD,), jnp.float32),
        pltpu.VMEM((NPT,), jnp.float32),
        pltpu.VMEM((NPT,), jnp.float32),
        pltpu.VMEM((NPT,), jnp.float32),
        pltpu.VMEM((NPT,), jnp.float32),
        pltpu.VMEM((NPT,), jnp.float32),
        pltpu.VMEM((NPT,), jnp.float32),
        pltpu.VMEM((NPT,), jnp.float32),
        pltpu.VMEM((NPT,), jnp.float32),
        pltpu.VMEM((CH,), jnp.int32),
        pltpu.VMEM((CH,), jnp.int32),
        pltpu.VMEM((CH,), jnp.int32),
        pltpu.VMEM((CH,), jnp.int32),
        pltpu.VMEM((CH,), jnp.int32),
        pltpu.VMEM((CH,), jnp.float32),
        pltpu.VMEM((CH,), jnp.float32),
        pltpu.SemaphoreType.DMA,
        pltpu.SemaphoreType.DMA,
        pltpu.SemaphoreType.DMA,
        pltpu.SemaphoreType.DMA,
        pltpu.SemaphoreType.DMA,
        pltpu.SemaphoreType.DMA,
        pltpu.SemaphoreType.DMA,
        pltpu.SemaphoreType.DMA,
        pltpu.SemaphoreType.DMA,
    ],
)(_scan_body)


@jax.jit
def kernel(x, params_phys, edge_index, W1, b1, W2, b2, W3, b3):
    dst_s, src_s = lax.sort((edge_index[1], edge_index[0]), num_keys=1)
    starts = jnp.searchsorted(dst_s, jnp.arange(16, dtype=jnp.int32) * NPT)
    probe = (dst_s[0] + src_s[-1] + starts[3]).astype(jnp.float32) * 0.0
    k16, a, b = _coeffs(params_phys, W1, b1, W2, b2, W3, b3)
    k16 = k16 + probe
    x_pad = jnp.pad(x, ((0, 0), (0, N_PAD - N)))
    src = jnp.pad(edge_index[0], (0, E_PAD - E), constant_values=N)
    dst = jnp.pad(edge_index[1], (0, E_PAD - E), constant_values=N)
    outs = _scan_kernel(x_pad, k16, a, b, src, dst)
    return outs[:, :N]


# speedup vs baseline: 2.4004x; 2.4004x over previous
import functools

import jax
import jax.numpy as jnp
from jax import lax
from jax.experimental import pallas as pl
from jax.experimental.pallas import tpu as pltpu
from jax.experimental.pallas import tpu_sc as plsc

N = 100000
E = 1600000
T = 50
HID = 256

NC = 2
NS = 16
NW = NC * NS
NPT = 3200
N_PAD = NW * NPT
EPT = 50176
E_PAD = NW * EPT
CH = 7168
NCH = EPT // CH
NV = NPT // 16
FW = 8

NB = 3200
N_GRID = N_PAD // NB


def _std_body(pT_ref, out_ref):
    x = pT_ref[...]
    s1 = jnp.sum(x, axis=1)
    s2 = jnp.sum(x * x, axis=1)
    out_ref[...] = jnp.pad(jnp.stack([s1, s2]), ((0, 0), (0, 125)))


def _col_stats(params_T):
    return pl.pallas_call(
        _std_body,
        out_shape=jax.ShapeDtypeStruct((2, 128), jnp.float32),
    )(params_T)


def _mlp_body(p_ref, w1_ref, b1_ref, w2_ref, b2_ref, w3_ref, b3_ref,
              stat_ref, k_ref, a_ref, b_ref):
    i = pl.program_id(0)
    s1 = stat_ref[0:1, 0:3] * (1.0 / N)
    s2 = stat_ref[1:2, 0:3] * (1.0 / N)
    stds = jnp.sqrt(jnp.maximum(s2 - s1 * s1, 0.0))
    lane = lax.broadcasted_iota(jnp.int32, (1, 3), 1)
    stds = jnp.where(lane == 0, 1.0, stds)
    w1s = w1_ref[...] / stds

    pn = p_ref[...]
    h = lax.dot_general(pn, w1s, (((1,), (1,)), ((), ())),
                        preferred_element_type=jnp.float32)
    h = jnp.maximum(h + b1_ref[...], 0.0)
    h = lax.dot_general(h, w2_ref[...], (((1,), (1,)), ((), ())),
                        preferred_element_type=jnp.float32)
    h = jnp.maximum(h + b2_ref[...], 0.0)
    raw = lax.dot_general(h, w3_ref[...], (((1,), (1,)), ((), ())),
                          preferred_element_type=jnp.float32)
    raw = raw + b3_ref[...]

    row = i * NB + lax.broadcasted_iota(jnp.int32, (NB, 1), 0)
    valid = row < N
    p0 = jax.nn.sigmoid(raw[:, 0:1])
    p1 = jax.nn.sigmoid(raw[:, 1:2] - 3.0)
    kk = (p0 * 0.25 + 0.005) * (1.0 / 16.0)
    xw = jnp.clip(p1 * 1.2, 0.0, 0.95)
    k_ref[...] = jnp.where(valid, kk, 0.0)
    a_ref[...] = jnp.where(valid, xw, 0.0)
    b_ref[...] = jnp.where(valid, 1.0 - xw, 0.0)


def _coeffs(params_phys, W1, b1, W2, b2, W3, b3):
    stats = _col_stats(params_phys.T)
    out_spec = pl.BlockSpec((NB, 1), lambda i: (i, 0))
    full = lambda *s: pl.BlockSpec(s, lambda i: tuple(0 for _ in s))
    k16, a, b = pl.pallas_call(
        _mlp_body,
        grid=(N_GRID,),
        in_specs=[
            pl.BlockSpec((NB, 3), lambda i: (i, 0)),
            full(HID, 3), full(1, HID), full(HID, HID), full(1, HID),
            full(2, HID), full(1, 2), full(2, 128),
        ],
        out_specs=[out_spec, out_spec, out_spec],
        out_shape=[jax.ShapeDtypeStruct((N_PAD, 1), jnp.float32)] * 3,
    )(params_phys, W1, b1.reshape(1, HID), W2, b2.reshape(1, HID),
      W3, b3.reshape(1, 2), stats)
    return k16.reshape(N_PAD), a.reshape(N_PAD), b.reshape(N_PAD)


def _scan_body(x_hbm, k_hbm, a_hbm, b_hbm, src_hbm, dst_hbm, flg_hbm,
               out_hbm, xfer_hbm, iflw_hbm,
               contrib, inflow, k16, av, bv, st, xb, cb, cb2, ib, ib2, zb,
               xsem_a, xsem_b,
               sbuf0, sbuf1, dbuf0, dbuf1, dbuf2, mbuf0, mbuf1,
               sem_s0, sem_s1, sem_d0, sem_d1, sem_d2, sem_g, sem_sc,
               sem_x, sem_o):
    sbuf = (sbuf0, sbuf1)
    dbuf = (dbuf0, dbuf1, dbuf2)
    mbuf = (mbuf0, mbuf1)
    sem_s = (sem_s0, sem_s1)
    sem_d = (sem_d0, sem_d1, sem_d2)
    c = lax.axis_index("c")
    s = lax.axis_index("s")
    w = c * NS + s
    wsib = (1 - c) * NS + s
    nbase = w * NPT
    ebase = w * EPT
    nsl = pl.ds(nbase, NPT)
    ssl = pl.ds(wsib * NPT, NPT)

    pltpu.sync_copy(k_hbm.at[nsl], k16)
    pltpu.sync_copy(a_hbm.at[nsl], av)
    pltpu.sync_copy(b_hbm.at[nsl], bv)

    def zero_i(i, carry):
        z = jnp.zeros((16,), jnp.float32)
        st[pl.ds(i * 16, 16)] = z
        cb[pl.ds(i * 16, 16)] = z
        zb[pl.ds(i * 16, 16)] = z
        ib2[pl.ds(i * 16, 16)] = z
        return carry

    lax.fori_loop(0, NV, zero_i, 0)

    def wflag(row, slot, val):
        sem = xsem_a if row == 0 else xsem_b

        @pl.when(val > 0)
        def _():
            pl.semaphore_signal(sem, 1, core_index=1 - c)

    def poll(row, slot, thr):
        sem = xsem_a if row == 0 else xsem_b
        @pl.when(thr > 0)
        def _():
            pl.semaphore_wait(sem, 1)

    def start_idx(t, i):
        eb = ebase + i * CH
        pltpu.async_copy(src_hbm.at[pl.ds(eb, CH)], sbuf[i % 2],
                         sem_s[i % 2])
        pltpu.async_copy(dst_hbm.at[pl.ds(eb, CH)], dbuf[i % 3],
                         sem_d[i % 3])

    def wait_idx(i):
        eb = pl.ds(0, CH)
        pltpu.make_async_copy(src_hbm.at[eb], sbuf[i % 2],
                              sem_s[i % 2]).wait()
        pltpu.make_async_copy(dst_hbm.at[eb], dbuf[i % 3],
                              sem_d[i % 3]).wait()

    def start_gather(i):
        return pltpu.async_copy(contrib.at[sbuf[i % 2]], mbuf[i % 2],
                                sem_g)

    def start_scatter(i):
        return pltpu.async_copy(mbuf[i % 2], inflow.at[dbuf[i % 3]],
                                sem_sc, add=True)

    def step(t, carry):
        start_idx(t, 0)
        start_idx(t, 1)
        cpx = pltpu.async_copy(x_hbm.at[t, nsl], xb, sem_x)
        pltpu.sync_copy(cb, contrib.at[nsl])
        pltpu.sync_copy(cb, xfer_hbm.at[nsl])
        wflag(0, w, t)

        pltpu.sync_copy(zb, inflow.at[nsl])
        pltpu.sync_copy(zb, inflow.at[ssl])

        poll(0, wsib, t)

        @pl.when(t > 0)
        def _():
            pltpu.sync_copy(xfer_hbm.at[ssl], cb2)
            pltpu.sync_copy(cb2, contrib.at[ssl])

        @pl.when(t == 0)
        def _():
            pltpu.sync_copy(zb, contrib.at[ssl])

        plsc.subcore_barrier()

        wait_idx(0)
        g = start_gather(0)
        sc = None
        for i in range(NCH):
            g.wait()
            if sc is not None:
                sc.wait()
            sc = start_scatter(i)
            if i + 1 < NCH:
                wait_idx(i + 1)
                g = start_gather(i + 1)
            if i + 2 < NCH:
                start_idx(t, i + 2)
        sc.wait()
        plsc.subcore_barrier()

        pltpu.sync_copy(inflow.at[ssl], ib2)
        pltpu.sync_copy(ib2, iflw_hbm.at[ssl])
        wflag(1, wsib, t)

        pltpu.sync_copy(inflow.at[nsl], ib)
        cpx.wait()

        poll(1, w, t)

        @pl.when(t > 0)
        def _():
            pltpu.sync_copy(iflw_hbm.at[nsl], ib2)

        @pl.when(t > 0)
        def _():
            pltpu.make_async_copy(st, out_hbm.at[t - 1, nsl], sem_o).wait()

        def update_i(i, c):
            for u in range(4):
                sl = pl.ds((i * 4 + u) * 16, 16)
                xt = xb[sl] * (1.0 / 86400.0)
                ns = av[sl] * st[sl] + bv[sl] * (ib[sl] + ib2[sl] + xt)
                st[sl] = ns
                cb[sl] = ns * k16[sl]
            return c

        lax.fori_loop(0, NV // 4, update_i, 0)
        pltpu.async_copy(st, out_hbm.at[t, nsl], sem_o)
        return carry

    lax.fori_loop(0, T, step, 0)
    pltpu.make_async_copy(st, out_hbm.at[T - 1, nsl], sem_o).wait()


_scan_kernel = functools.partial(
    pl.kernel,
    out_type=[
        jax.ShapeDtypeStruct((T, N_PAD), jnp.float32),
        jax.ShapeDtypeStruct((N_PAD,), jnp.float32),
        jax.ShapeDtypeStruct((N_PAD,), jnp.float32),
    ],
    mesh=plsc.VectorSubcoreMesh(core_axis_name="c", subcore_axis_name="s"),
    scratch_types=[
        pltpu.VMEM_SHARED((N_PAD,), jnp.float32),
        pltpu.VMEM_SHARED((N_PAD,), jnp.float32),
        pltpu.VMEM((NPT,), jnp.float32),
        pltpu.VMEM((NPT,), jnp.float32),
        pltpu.VMEM((NPT,), jnp.float32),
        pltpu.VMEM((NPT,), jnp.float32),
        pltpu.VMEM((NPT,), jnp.float32),
        pltpu.VMEM((NPT,), jnp.float32),
        pltpu.VMEM((NPT,), jnp.float32),
        pltpu.VMEM((NPT,), jnp.float32),
        pltpu.VMEM((NPT,), jnp.float32),
        pltpu.VMEM((NPT,), jnp.float32),
        pltpu.SemaphoreType.REGULAR,
        pltpu.SemaphoreType.REGULAR,
        pltpu.VMEM((CH,), jnp.int32),
        pltpu.VMEM((CH,), jnp.int32),
        pltpu.VMEM((CH,), jnp.int32),
        pltpu.VMEM((CH,), jnp.int32),
        pltpu.VMEM((CH,), jnp.int32),
        pltpu.VMEM((CH,), jnp.float32),
        pltpu.VMEM((CH,), jnp.float32),
        pltpu.SemaphoreType.DMA,
        pltpu.SemaphoreType.DMA,
        pltpu.SemaphoreType.DMA,
        pltpu.SemaphoreType.DMA,
        pltpu.SemaphoreType.DMA,
        pltpu.SemaphoreType.DMA,
        pltpu.SemaphoreType.DMA,
        pltpu.SemaphoreType.DMA,
        pltpu.SemaphoreType.DMA,
    ],
)(_scan_body)


@jax.jit
def kernel(x, params_phys, edge_index, W1, b1, W2, b2, W3, b3):
    k16, a, b = _coeffs(params_phys, W1, b1, W2, b2, W3, b3)
    x_pad = jnp.pad(x, ((0, 0), (0, N_PAD - N)))
    src = jnp.pad(edge_index[0], (0, E_PAD - E), constant_values=N)
    dst = jnp.pad(edge_index[1], (0, E_PAD - E), constant_values=N)
    flg = (b[:NW * FW * 2] * 0.0).astype(jnp.int32)
    outs, _, _ = _scan_kernel(x_pad, k16, a, b, src, dst, flg)
    return outs[:, :N]
